# baseline (speedup 1.0000x reference)
import functools

import jax
import jax.numpy as jnp
from jax.experimental import pallas as pl
from jax.experimental.pallas import tpu as pltpu

B, T = 4, 32
C, H, W = 96, 56, 56
N = C * H * W
LANES = 128
SUB = N // LANES
BT = B * T
K = 8
ROWS_PER_STEP = 8


def _scores_body(x_ref, o_ref):
    s = jnp.sum(jnp.abs(x_ref[...]), axis=(1, 2)) * (1.0 / N)
    o_ref[...] = s.reshape(ROWS_PER_STEP, 1)


def _mask_body(s_ref, mask_ref, src_ref):
    scores = s_ref[...]
    col = jax.lax.broadcasted_iota(jnp.int32, (B, T), 1)
    sel = jnp.zeros((B, T), dtype=jnp.bool_)
    work = scores
    neg = jnp.float32(-jnp.inf)
    for _ in range(K):
        m = jnp.max(work, axis=1, keepdims=True)
        cand = work == m
        first_idx = jnp.min(jnp.where(cand, col, T), axis=1, keepdims=True)
        first = col == first_idx
        sel = jnp.logical_or(sel, first)
        work = jnp.where(first, neg, work)
    mask_ref[...] = sel.astype(jnp.int32)
    c = jnp.where(sel, col, -1)
    for sh in (1, 2, 4, 8, 16):
        shifted = jnp.concatenate(
            [jnp.full((B, sh), -1, jnp.int32), c[:, : T - sh]], axis=1)
        c = jnp.maximum(c, shifted)
    first_kept = jnp.min(jnp.where(sel, col, T), axis=1, keepdims=True)
    src = jnp.where(c < 0, first_kept, c)
    row = jax.lax.broadcasted_iota(jnp.int32, (B, T), 0)
    src_ref[...] = src + row * T


def _write_body(src_ref, keep_ref, x_ref, o_ref):
    i = pl.program_id(0)

    @pl.when(keep_ref[i] == 1)
    def _():
        o_ref[...] = x_ref[...]

    @pl.when(keep_ref[i] == 0)
    def _():
        o_ref[...] = jnp.zeros_like(o_ref)


@jax.jit
def kernel(x):
    x3 = x.reshape(BT, SUB, LANES)

    scores_row = pl.pallas_call(
        _scores_body,
        grid=(BT // ROWS_PER_STEP,),
        in_specs=[pl.BlockSpec((ROWS_PER_STEP, SUB, LANES),
                               lambda i: (i, 0, 0))],
        out_specs=pl.BlockSpec((ROWS_PER_STEP, 1), lambda i: (i, 0)),
        out_shape=jax.ShapeDtypeStruct((BT, 1), jnp.float32),
    )(x3)

    scores = scores_row.reshape(B, T)

    mask_i32, src = pl.pallas_call(
        _mask_body,
        out_shape=(jax.ShapeDtypeStruct((B, T), jnp.int32),
                   jax.ShapeDtypeStruct((B, T), jnp.int32)),
    )(scores)

    src_flat = src.reshape(BT)
    keep_flat = mask_i32.reshape(BT)

    out3 = pl.pallas_call(
        _write_body,
        grid_spec=pltpu.PrefetchScalarGridSpec(
            num_scalar_prefetch=2,
            grid=(BT,),
            in_specs=[pl.BlockSpec((1, SUB, LANES),
                                   lambda i, src_r, keep_r: (src_r[i], 0, 0))],
            out_specs=pl.BlockSpec((1, SUB, LANES),
                                   lambda i, src_r, keep_r: (i, 0, 0)),
        ),
        out_shape=jax.ShapeDtypeStruct((BT, SUB, LANES), jnp.float32),
    )(src_flat, keep_flat, x3)

    masked = out3.reshape(B, T, C, H, W)
    return masked, mask_i32.astype(jnp.bool_)


# speedup vs baseline: 6.9417x; 6.9417x over previous
import jax
import jax.numpy as jnp
from jax.experimental import pallas as pl
from jax.experimental.pallas import tpu as pltpu

B, T = 4, 32
C, H, W = 96, 56, 56
HW = H * W
N = C * HW
BT = B * T
K = 8
ROWS_PER_STEP = 4


def _scores_body(x_ref, o_ref):
    s = jnp.sum(jnp.abs(x_ref[...]), axis=(1, 2)) * (1.0 / N)
    o_ref[...] = s.reshape(1, 1, ROWS_PER_STEP)


def _mask_body(s_ref, mask_ref, src_ref):
    scores = s_ref[...]
    col = jax.lax.broadcasted_iota(jnp.int32, (B, T), 1)
    sel = jnp.zeros((B, T), dtype=jnp.bool_)
    work = scores
    neg = jnp.float32(-jnp.inf)
    for _ in range(K):
        m = jnp.max(work, axis=1, keepdims=True)
        cand = work == m
        first_idx = jnp.min(jnp.where(cand, col, T), axis=1, keepdims=True)
        first = col == first_idx
        sel = jnp.logical_or(sel, first)
        work = jnp.where(first, neg, work)
    mask_ref[...] = sel.astype(jnp.int32)
    c = jnp.where(sel, col, -1)
    for sh in (1, 2, 4, 8, 16):
        shifted = jnp.concatenate(
            [jnp.full((B, sh), -1, jnp.int32), c[:, : T - sh]], axis=1)
        c = jnp.maximum(c, shifted)
    first_kept = jnp.min(jnp.where(sel, col, T), axis=1, keepdims=True)
    src = jnp.where(c < 0, first_kept, c)
    row = jax.lax.broadcasted_iota(jnp.int32, (B, T), 0)
    src_ref[...] = src + row * T


def _write_body(src_ref, keep_ref, x_ref, o_ref):
    i = pl.program_id(0)

    @pl.when(keep_ref[i] == 1)
    def _():
        o_ref[...] = x_ref[...]

    @pl.when(keep_ref[i] == 0)
    def _():
        o_ref[...] = jnp.zeros_like(o_ref)


@jax.jit
def kernel(x):
    xv = x.transpose(0, 1, 3, 4, 2).reshape(BT, HW, C)

    scores_col = pl.pallas_call(
        _scores_body,
        grid=(BT // ROWS_PER_STEP,),
        in_specs=[pl.BlockSpec((ROWS_PER_STEP, HW, C), lambda i: (i, 0, 0))],
        out_specs=pl.BlockSpec((1, 1, ROWS_PER_STEP), lambda i: (i, 0, 0)),
        out_shape=jax.ShapeDtypeStruct((BT // ROWS_PER_STEP, 1, ROWS_PER_STEP),
                                       jnp.float32),
    )(xv)

    scores = scores_col.reshape(B, T)

    mask_i32, src = pl.pallas_call(
        _mask_body,
        out_shape=(jax.ShapeDtypeStruct((B, T), jnp.int32),
                   jax.ShapeDtypeStruct((B, T), jnp.int32)),
    )(scores)

    src_flat = src.reshape(BT)
    keep_flat = mask_i32.reshape(BT)

    outv = pl.pallas_call(
        _write_body,
        grid_spec=pltpu.PrefetchScalarGridSpec(
            num_scalar_prefetch=2,
            grid=(BT,),
            in_specs=[pl.BlockSpec((1, HW, C),
                                   lambda i, src_r, keep_r: (src_r[i], 0, 0))],
            out_specs=pl.BlockSpec((1, HW, C),
                                   lambda i, src_r, keep_r: (i, 0, 0)),
        ),
        out_shape=jax.ShapeDtypeStruct((BT, HW, C), jnp.float32),
    )(src_flat, keep_flat, xv)

    masked = outv.reshape(B, T, H, W, C).transpose(0, 1, 4, 2, 3)
    return masked, mask_i32.astype(jnp.bool_)


# speedup vs baseline: 7.6883x; 1.1076x over previous
import jax
import jax.numpy as jnp
from jax.experimental import pallas as pl
from jax.experimental.pallas import tpu as pltpu

B, T = 4, 32
C, H, W = 96, 56, 56
HW = H * W
N = C * HW
BT = B * T
K = 8
NKEPT = B * K
ROWS_PER_STEP = 4


def _scores_body(x_ref, o_ref):
    s = jnp.sum(jnp.abs(x_ref[...]), axis=(1, 2)) * (1.0 / N)
    o_ref[...] = s.reshape(1, 1, ROWS_PER_STEP)


def _mask_body(s_ref, mask_ref, step_ref):
    scores = s_ref[...]
    col = jax.lax.broadcasted_iota(jnp.int32, (B, T), 1)
    sel = jnp.zeros((B, T), dtype=jnp.bool_)
    work = scores
    neg = jnp.float32(-jnp.inf)
    for _ in range(K):
        m = jnp.max(work, axis=1, keepdims=True)
        cand = work == m
        first_idx = jnp.min(jnp.where(cand, col, T), axis=1, keepdims=True)
        first = col == first_idx
        sel = jnp.logical_or(sel, first)
        work = jnp.where(first, neg, work)
    mask_ref[...] = sel.astype(jnp.int32)
    cs = sel.astype(jnp.int32)
    for sh in (1, 2, 4, 8, 16):
        shifted = jnp.concatenate(
            [jnp.zeros((B, sh), jnp.int32), cs[:, : T - sh]], axis=1)
        cs = cs + shifted
    row = jax.lax.broadcasted_iota(jnp.int32, (B, T), 0)
    csp = (col + 1) - cs
    step_kept = row * K + cs - 1
    step_pruned = NKEPT + row * (T - K) + csp - 1
    step_ref[...] = jnp.where(sel, step_kept, step_pruned)


def _write_body(srcp_ref, outp_ref, x_ref, o_ref):
    j = pl.program_id(0)

    @pl.when(j < NKEPT)
    def _():
        o_ref[...] = x_ref[...]

    @pl.when(j >= NKEPT)
    def _():
        o_ref[...] = jnp.zeros_like(o_ref)


@jax.jit
def kernel(x):
    xv = x.transpose(0, 1, 3, 4, 2).reshape(BT, HW, C)

    scores_col = pl.pallas_call(
        _scores_body,
        grid=(BT // ROWS_PER_STEP,),
        in_specs=[pl.BlockSpec((ROWS_PER_STEP, HW, C), lambda i: (i, 0, 0))],
        out_specs=pl.BlockSpec((1, 1, ROWS_PER_STEP), lambda i: (i, 0, 0)),
        out_shape=jax.ShapeDtypeStruct((BT // ROWS_PER_STEP, 1, ROWS_PER_STEP),
                                       jnp.float32),
    )(xv)

    scores = scores_col.reshape(B, T)

    mask_i32, stepidx = pl.pallas_call(
        _mask_body,
        out_shape=(jax.ShapeDtypeStruct((B, T), jnp.int32),
                   jax.ShapeDtypeStruct((B, T), jnp.int32)),
    )(scores)

    step_flat = stepidx.reshape(BT)
    ar = jnp.arange(BT, dtype=jnp.int32)
    outp = jnp.sum((step_flat[:, None] == ar[None, :]) * ar[:, None],
                   axis=0, dtype=jnp.int32)
    srcp = jnp.where(ar < NKEPT, outp, outp[NKEPT - 1])

    outv = pl.pallas_call(
        _write_body,
        grid_spec=pltpu.PrefetchScalarGridSpec(
            num_scalar_prefetch=2,
            grid=(BT,),
            in_specs=[pl.BlockSpec((1, HW, C),
                                   lambda j, srcp_r, outp_r: (srcp_r[j], 0, 0))],
            out_specs=pl.BlockSpec((1, HW, C),
                                   lambda j, srcp_r, outp_r: (outp_r[j], 0, 0)),
        ),
        out_shape=jax.ShapeDtypeStruct((BT, HW, C), jnp.float32),
    )(srcp, outp, xv)

    masked = outv.reshape(B, T, H, W, C).transpose(0, 1, 4, 2, 3)
    return masked, mask_i32.astype(jnp.bool_)


# speedup vs baseline: 7.6936x; 1.0007x over previous
import jax
import jax.numpy as jnp
from jax.experimental import pallas as pl
from jax.experimental.pallas import tpu as pltpu

B, T = 4, 32
C, H, W = 96, 56, 56
HW = H * W
N = C * HW
BT = B * T
K = 8
ROWS_PER_STEP = 4


def _scores_body(x_ref, o_ref):
    s = jnp.sum(jnp.abs(x_ref[...]), axis=(1, 2)) * (1.0 / N)
    o_ref[...] = s.reshape(1, 1, ROWS_PER_STEP)


def _mask_body(s_ref, mask_ref, out_sched_ref, src_sched_ref):
    scores = s_ref[...]
    col = jax.lax.broadcasted_iota(jnp.int32, (B, T), 1)
    sel = jnp.zeros((B, T), dtype=jnp.bool_)
    work = scores
    neg = jnp.float32(-jnp.inf)
    for _ in range(K):
        m = jnp.max(work, axis=1, keepdims=True)
        cand = work == m
        first_idx = jnp.min(jnp.where(cand, col, T), axis=1, keepdims=True)
        first = col == first_idx
        sel = jnp.logical_or(sel, first)
        work = jnp.where(first, neg, work)
    mask_ref[...] = sel.astype(jnp.int32)
    cs = sel.astype(jnp.int32)
    for sh in (1, 2, 4, 8, 16):
        shifted = jnp.concatenate(
            [jnp.zeros((B, sh), jnp.int32), cs[:, : T - sh]], axis=1)
        cs = cs + shifted
    csp = (col + 1) - cs
    kept = [jnp.min(jnp.where(sel & (cs == r + 1), col, T), axis=1,
                    keepdims=True) for r in range(K)]
    pruned = [jnp.min(jnp.where((~sel) & (csp == r + 1), col, T), axis=1,
                      keepdims=True) for r in range(T - K)]
    out_sched = jnp.concatenate(kept + pruned, axis=1)
    last_kept = kept[K - 1]
    src_sched = jnp.concatenate(kept + [last_kept] * (T - K), axis=1)
    row = jax.lax.broadcasted_iota(jnp.int32, (B, T), 0)
    out_sched_ref[...] = out_sched + row * T
    src_sched_ref[...] = src_sched + row * T


def _write_body(src_ref, out_ref_tab, x_ref, o_ref):
    j = pl.program_id(0)

    @pl.when(jax.lax.rem(j, T) < K)
    def _():
        o_ref[...] = x_ref[...]

    @pl.when(jax.lax.rem(j, T) >= K)
    def _():
        o_ref[...] = jnp.zeros_like(o_ref)


@jax.jit
def kernel(x):
    xv = x.transpose(0, 1, 3, 4, 2).reshape(BT, HW, C)

    scores_col = pl.pallas_call(
        _scores_body,
        grid=(BT // ROWS_PER_STEP,),
        in_specs=[pl.BlockSpec((ROWS_PER_STEP, HW, C), lambda i: (i, 0, 0))],
        out_specs=pl.BlockSpec((1, 1, ROWS_PER_STEP), lambda i: (i, 0, 0)),
        out_shape=jax.ShapeDtypeStruct((BT // ROWS_PER_STEP, 1, ROWS_PER_STEP),
                                       jnp.float32),
    )(xv)

    scores = scores_col.reshape(B, T)

    mask_i32, out_sched, src_sched = pl.pallas_call(
        _mask_body,
        out_shape=(jax.ShapeDtypeStruct((B, T), jnp.int32),
                   jax.ShapeDtypeStruct((B, T), jnp.int32),
                   jax.ShapeDtypeStruct((B, T), jnp.int32)),
    )(scores)

    outv = pl.pallas_call(
        _write_body,
        grid_spec=pltpu.PrefetchScalarGridSpec(
            num_scalar_prefetch=2,
            grid=(BT,),
            in_specs=[pl.BlockSpec(
                (1, HW, C),
                lambda j, src_r, out_r: (src_r[j // T, jax.lax.rem(j, T)],
                                         0, 0))],
            out_specs=pl.BlockSpec(
                (1, HW, C),
                lambda j, src_r, out_r: (out_r[j // T, jax.lax.rem(j, T)],
                                         0, 0)),
        ),
        out_shape=jax.ShapeDtypeStruct((BT, HW, C), jnp.float32),
    )(src_sched, out_sched, xv)

    masked = outv.reshape(B, T, H, W, C).transpose(0, 1, 4, 2, 3)
    return masked, mask_i32.astype(jnp.bool_)


# speedup vs baseline: 8.6452x; 1.1237x over previous
import jax
import jax.numpy as jnp
from jax.experimental import pallas as pl
from jax.experimental.pallas import tpu as pltpu

B, T = 4, 32
C, H, W = 96, 56, 56
HW = H * W
N = C * HW
BT = B * T
K = 8
ROWS_PER_STEP = 4
G = 4


def _scores_body(x_ref, o_ref):
    s = jnp.sum(jnp.abs(x_ref[...]), axis=(1, 2)) * (1.0 / N)
    o_ref[...] = s.reshape(1, 1, ROWS_PER_STEP)


def _mask_body(s_ref, mask_ref):
    scores = s_ref[...]
    col = jax.lax.broadcasted_iota(jnp.int32, (B, T), 1)
    sel = jnp.zeros((B, T), dtype=jnp.bool_)
    work = scores
    neg = jnp.float32(-jnp.inf)
    for _ in range(K):
        m = jnp.max(work, axis=1, keepdims=True)
        cand = work == m
        first_idx = jnp.min(jnp.where(cand, col, T), axis=1, keepdims=True)
        first = col == first_idx
        sel = jnp.logical_or(sel, first)
        work = jnp.where(first, neg, work)
    mask_ref[...] = sel.astype(jnp.int32)


def _write_body(src_ref, keep_ref, x0_ref, x1_ref, x2_ref, x3_ref, o_ref):
    g = pl.program_id(0)
    x_refs = (x0_ref, x1_ref, x2_ref, x3_ref)
    for s in range(G):
        @pl.when(keep_ref[g, s] == 1)
        def _(s=s):
            o_ref[s, :, :] = x_refs[s][0, :, :]

        @pl.when(keep_ref[g, s] == 0)
        def _(s=s):
            o_ref[s, :, :] = jnp.zeros((HW, C), jnp.float32)


@jax.jit
def kernel(x):
    xv = x.transpose(0, 1, 3, 4, 2).reshape(BT, HW, C)

    scores_col = pl.pallas_call(
        _scores_body,
        grid=(BT // ROWS_PER_STEP,),
        in_specs=[pl.BlockSpec((ROWS_PER_STEP, HW, C), lambda i: (i, 0, 0))],
        out_specs=pl.BlockSpec((1, 1, ROWS_PER_STEP), lambda i: (i, 0, 0)),
        out_shape=jax.ShapeDtypeStruct((BT // ROWS_PER_STEP, 1, ROWS_PER_STEP),
                                       jnp.float32),
    )(xv)

    scores = scores_col.reshape(B, T)

    mask_i32 = pl.pallas_call(
        _mask_body,
        out_shape=jax.ShapeDtypeStruct((B, T), jnp.int32),
    )(scores)

    keep_flat = mask_i32.reshape(BT)
    k4 = keep_flat.reshape(BT // G, G)
    ar4 = jnp.arange(BT, dtype=jnp.int32).reshape(BT // G, G)
    marked = jnp.where(k4 == 1, ar4, -1)
    prev = jax.lax.cummax(marked, axis=0)
    first = jnp.min(jnp.where(k4 == 1, ar4, BT), axis=0, keepdims=True)
    src4 = jnp.where(prev < 0, first, prev).astype(jnp.int32)

    outv = pl.pallas_call(
        _write_body,
        grid_spec=pltpu.PrefetchScalarGridSpec(
            num_scalar_prefetch=2,
            grid=(BT // G,),
            in_specs=[
                pl.BlockSpec((1, HW, C),
                             lambda g, src_r, keep_r, s=s: (src_r[g, s], 0, 0))
                for s in range(G)
            ],
            out_specs=pl.BlockSpec((G, HW, C),
                                   lambda g, src_r, keep_r: (g, 0, 0)),
        ),
        out_shape=jax.ShapeDtypeStruct((BT, HW, C), jnp.float32),
    )(src4, k4, xv, xv, xv, xv)

    masked = outv.reshape(B, T, H, W, C).transpose(0, 1, 4, 2, 3)
    return masked, mask_i32.astype(jnp.bool_)


# speedup vs baseline: 9.0541x; 1.0473x over previous
import jax
import jax.numpy as jnp
from jax.experimental import pallas as pl
from jax.experimental.pallas import tpu as pltpu

B, T = 4, 32
C, H, W = 96, 56, 56
HW = H * W
N = C * HW
BT = B * T
K = 8
ROWS_PER_STEP = 8
G = 8


def _scores_body(x_ref, o_ref):
    s = jnp.sum(jnp.abs(x_ref[...]), axis=(1, 2)) * (1.0 / N)
    o_ref[...] = s.reshape(1, 1, ROWS_PER_STEP)


def _mask_body(s_ref, mask_ref):
    scores = s_ref[...]
    col = jax.lax.broadcasted_iota(jnp.int32, (B, T), 1)
    sel = jnp.zeros((B, T), dtype=jnp.bool_)
    work = scores
    neg = jnp.float32(-jnp.inf)
    for _ in range(K):
        m = jnp.max(work, axis=1, keepdims=True)
        cand = work == m
        first_idx = jnp.min(jnp.where(cand, col, T), axis=1, keepdims=True)
        first = col == first_idx
        sel = jnp.logical_or(sel, first)
        work = jnp.where(first, neg, work)
    mask_ref[...] = sel.astype(jnp.int32)


def _write_body(src_ref, keep_ref, *refs):
    g = pl.program_id(0)
    x_refs = refs[:G]
    o_ref = refs[G]
    for s in range(G):
        @pl.when(keep_ref[g, s] == 1)
        def _(s=s):
            o_ref[s, :, :] = x_refs[s][0, :, :]

        @pl.when(keep_ref[g, s] == 0)
        def _(s=s):
            o_ref[s, :, :] = jnp.zeros((HW, C), jnp.float32)


@jax.jit
def kernel(x):
    xv = x.transpose(0, 1, 3, 4, 2).reshape(BT, HW, C)

    scores_col = pl.pallas_call(
        _scores_body,
        grid=(BT // ROWS_PER_STEP,),
        in_specs=[pl.BlockSpec((ROWS_PER_STEP, HW, C), lambda i: (i, 0, 0))],
        out_specs=pl.BlockSpec((1, 1, ROWS_PER_STEP), lambda i: (i, 0, 0)),
        out_shape=jax.ShapeDtypeStruct((BT // ROWS_PER_STEP, 1, ROWS_PER_STEP),
                                       jnp.float32),
    )(xv)

    scores = scores_col.reshape(B, T)

    mask_i32 = pl.pallas_call(
        _mask_body,
        out_shape=jax.ShapeDtypeStruct((B, T), jnp.int32),
    )(scores)

    keep_flat = mask_i32.reshape(BT)
    k4 = keep_flat.reshape(BT // G, G)
    ar4 = jnp.arange(BT, dtype=jnp.int32).reshape(BT // G, G)
    marked = jnp.where(k4 == 1, ar4, -1)
    prev = jax.lax.cummax(marked, axis=0)
    first = jnp.min(jnp.where(k4 == 1, ar4, BT), axis=0, keepdims=True)
    src4 = jnp.where(prev < 0, first, prev).astype(jnp.int32)

    outv = pl.pallas_call(
        _write_body,
        grid_spec=pltpu.PrefetchScalarGridSpec(
            num_scalar_prefetch=2,
            grid=(BT // G,),
            in_specs=[
                pl.BlockSpec((1, HW, C),
                             lambda g, src_r, keep_r, s=s: (src_r[g, s], 0, 0))
                for s in range(G)
            ],
            out_specs=pl.BlockSpec((G, HW, C),
                                   lambda g, src_r, keep_r: (g, 0, 0)),
        ),
        out_shape=jax.ShapeDtypeStruct((BT, HW, C), jnp.float32),
    )(src4, k4, *([xv] * G))

    masked = outv.reshape(B, T, H, W, C).transpose(0, 1, 4, 2, 3)
    return masked, mask_i32.astype(jnp.bool_)
